# baseline (speedup 1.0000x reference)
import functools

import jax
import jax.numpy as jnp
import numpy as np
from jax import lax
from jax.experimental import pallas as pl
from jax.experimental.pallas import tpu as pltpu
from jax.experimental.pallas import tpu_sc as plsc

N_NODES = 10000
E = 160000
F = 256
H = 128
NC = 2
NS = 16
C = 64
NCHUNK = 162
EPS = NCHUNK * C
E_PAD = NS * EPS
N_PAD = 10240
RPS = N_PAD // NS
GROUPS = C // 16

_PERM = np.empty((F,), np.int64)
for _t in range(F // 32):
    _base = _t * 32
    for _i in range(16):
        _PERM[_base + _i] = _base + 2 * _i
        _PERM[_base + 16 + _i] = _base + 2 * _i + 1


def _sc_body(featb2, rows2, colp, ewp, g_out,
             rv, rb0, rb1, rb2, ms0, ms1, ms2, col0, col1, col2,
             wv0, wv1, wv2, acc, semg, sems, semc, semw):
    c = lax.axis_index("c")
    s = lax.axis_index("s")
    rings = (
        (rb0, ms0, col0, wv0),
        (rb1, ms1, col1, wv1),
        (rb2, ms2, col2, wv2),
    )

    pltpu.sync_copy(rows2.at[c, pl.ds(s * EPS, EPS)], rv)

    def zrow(i, carry):
        for t in range(H // 16):
            ms0[i, pl.ds(t * 16, 16)] = jnp.zeros((16,), jnp.float32)
        return carry

    lax.fori_loop(0, C, zrow, 0)
    stripe = s * RPS
    for i in range(RPS // C):
        pltpu.sync_copy(ms0, acc.at[pl.ds(stripe + i * C, C)])
    plsc.subcore_barrier()

    ebase = s * EPS

    def gather_start(j, r, rbuf):
        pltpu.async_copy(featb2.at[rv.at[pl.ds(j * C, C)]], rbuf, semg.at[r])

    def gather_wait(j, r, rbuf):
        pltpu.make_async_copy(
            featb2.at[rv.at[pl.ds(j * C, C)]], rbuf, semg.at[r]
        ).wait()

    def colw_start(j, r, cbuf, wbuf):
        pltpu.async_copy(colp.at[pl.ds(ebase + j * C, C)], cbuf, semc.at[r])
        pltpu.async_copy(ewp.at[pl.ds(ebase + j * C, C)], wbuf, semw.at[r])

    def colw_wait(j, r, cbuf, wbuf):
        pltpu.make_async_copy(
            colp.at[pl.ds(ebase + j * C, C)], cbuf, semc.at[r]
        ).wait()
        pltpu.make_async_copy(
            ewp.at[pl.ds(ebase + j * C, C)], wbuf, semw.at[r]
        ).wait()

    def scatter_start(r, mbuf, cbuf):
        pltpu.async_copy(mbuf, acc.at[cbuf], sems.at[r], add=True)

    def scatter_wait(r, mbuf, cbuf):
        pltpu.make_async_copy(mbuf, acc.at[cbuf], sems.at[r]).wait()

    colw_start(0, 0, col0, wv0)
    gather_start(0, 0, rb0)

    def do_slot(j, b):
        rbuf, mbuf, cbuf, wbuf = rings[b]
        nb = (b + 1) % 3
        nrbuf, nmbuf, ncbuf, nwbuf = rings[nb]

        @pl.when(j >= 2)
        def _():
            scatter_wait(nb, nmbuf, ncbuf)

        @pl.when(j + 1 < NCHUNK)
        def _():
            colw_start(j + 1, nb, ncbuf, nwbuf)
            gather_start(j + 1, nb, nrbuf)

        gather_wait(j, b, rbuf)
        colw_wait(j, b, cbuf, wbuf)

        def emul(g, carry2):
            wvec = wbuf[pl.ds(g * 16, 16)]
            himask = jnp.full((16,), -65536, jnp.int32)
            for lane in range(16):
                bidx = jnp.full((16,), lane, jnp.int32)
                wb = wvec.at[bidx].get(mode="promise_in_bounds")
                e = g * 16 + lane
                for t in range(H // 32):
                    pi = rbuf[e, pl.ds(t * 16, 16)]
                    lo = lax.bitcast_convert_type(pi << 16, jnp.float32)
                    hi = lax.bitcast_convert_type(pi & himask, jnp.float32)
                    mbuf[e, pl.ds(t * 32, 16)] = lo * wb
                    mbuf[e, pl.ds(t * 32 + 16, 16)] = hi * wb
            return carry2

        lax.fori_loop(0, GROUPS, emul, 0)

        scatter_start(b, mbuf, cbuf)

    def triple(p, carry):
        j0 = p * 3
        do_slot(j0, 0)
        do_slot(j0 + 1, 1)
        do_slot(j0 + 2, 2)
        return carry

    lax.fori_loop(0, NCHUNK // 3, triple, 0)
    scatter_wait(1, ms1, col1)
    scatter_wait(2, ms2, col2)
    plsc.subcore_barrier()

    for i in range(RPS // C):
        r0 = stripe + i * C
        pltpu.sync_copy(
            acc.at[pl.ds(r0, C)], g_out.at[pl.ds(r0, C), pl.ds(c * H, H)]
        )


_sc_scatter = functools.partial(
    pl.kernel,
    mesh=plsc.VectorSubcoreMesh(core_axis_name="c", subcore_axis_name="s"),
    out_type=jax.ShapeDtypeStruct((N_PAD, F), jnp.float32),
    compiler_params=pltpu.CompilerParams(use_tc_tiling_on_sc=False),
    scratch_types=[
        pltpu.VMEM((EPS,), jnp.int32),
        pltpu.VMEM((C, H // 2), jnp.int32),
        pltpu.VMEM((C, H // 2), jnp.int32),
        pltpu.VMEM((C, H // 2), jnp.int32),
        pltpu.VMEM((C, H), jnp.float32),
        pltpu.VMEM((C, H), jnp.float32),
        pltpu.VMEM((C, H), jnp.float32),
        pltpu.VMEM((C,), jnp.int32),
        pltpu.VMEM((C,), jnp.int32),
        pltpu.VMEM((C,), jnp.int32),
        pltpu.VMEM((C,), jnp.float32),
        pltpu.VMEM((C,), jnp.float32),
        pltpu.VMEM((C,), jnp.float32),
        pltpu.VMEM_SHARED((N_PAD, H), jnp.float32),
        pltpu.SemaphoreType.DMA((3,)),
        pltpu.SemaphoreType.DMA((3,)),
        pltpu.SemaphoreType.DMA((3,)),
        pltpu.SemaphoreType.DMA((3,)),
    ],
)(_sc_body)


def _mm_body(g_ref, wt_ref, a_ref, o_ref):
    x = jnp.dot(g_ref[...], wt_ref[...], preferred_element_type=jnp.float32)
    a = a_ref[0]
    o_ref[...] = jnp.where(x >= 0.0, x, a * x)


def _matmul_prelu(g, wt, a_arr):
    return pl.pallas_call(
        _mm_body,
        grid=(N_PAD // 1024,),
        in_specs=[
            pl.BlockSpec((1024, F), lambda i: (i, 0)),
            pl.BlockSpec((F, F), lambda i: (0, 0)),
            pl.BlockSpec(memory_space=pltpu.SMEM),
        ],
        out_specs=pl.BlockSpec((1024, F), lambda i: (i, 0)),
        out_shape=jax.ShapeDtypeStruct((N_PAD, F), jnp.float32),
    )(g, wt, a_arr)


@jax.jit
def kernel(feat, edge_index, edge_weight, W, prelu_a):
    row = edge_index[0].astype(jnp.int32)
    col = edge_index[1].astype(jnp.int32)
    zpad = jnp.zeros((E_PAD - E,), jnp.int32)
    rowp = jnp.concatenate([row, zpad])
    colp = jnp.concatenate([col, zpad])
    ewp = jnp.concatenate([edge_weight, jnp.zeros((E_PAD - E,), jnp.float32)])
    rows2 = jnp.stack([rowp, rowp + N_NODES])
    featb = feat.astype(jnp.bfloat16)
    featb2 = jnp.concatenate([featb[:, :H], featb[:, H:]], axis=0)
    featb2 = lax.bitcast_convert_type(
        featb2.reshape(2 * N_NODES, H // 2, 2), jnp.int32
    )

    g = _sc_scatter(featb2, rows2, colp, ewp)

    wt_perm = W.T[jnp.asarray(_PERM), :]
    out = _matmul_prelu(g, wt_perm, prelu_a.reshape(1))
    return out[:N_NODES]


# speedup vs baseline: 1.3367x; 1.3367x over previous
import functools

import jax
import jax.numpy as jnp
import numpy as np
from jax import lax
from jax.experimental import pallas as pl
from jax.experimental.pallas import tpu as pltpu
from jax.experimental.pallas import tpu_sc as plsc

N_NODES = 10000
E = 160000
F = 256
H = 128
NC = 2
NS = 16
C = 64
NCHUNK = 162
EPS = NCHUNK * C
E_PAD = NS * EPS
N_PAD = 10240
RPS = N_PAD // NS
GROUPS = C // 16

_PERM = np.empty((F,), np.int64)
for _h in range(2):
    for _t in range(4):
        for _i in range(16):
            _PERM[_h * 128 + _t * 32 + _i] = _h * 128 + _t * 16 + _i
            _PERM[_h * 128 + _t * 32 + 16 + _i] = _h * 128 + _t * 16 + _i + 64


def _sc_body(featb2, rows2, colp, ewp, g_out,
             rv, rb0, rb1, rb2, ms0, ms1, ms2, col0, col1, col2,
             wv0, wv1, wv2, acc, semg, sems, semc, semw):
    c = lax.axis_index("c")
    s = lax.axis_index("s")
    rings = (
        (rb0, ms0, col0, wv0),
        (rb1, ms1, col1, wv1),
        (rb2, ms2, col2, wv2),
    )

    pltpu.sync_copy(rows2.at[c, pl.ds(s * EPS, EPS)], rv)

    def zrow(i, carry):
        for t in range(H // 16):
            ms0[i, pl.ds(t * 16, 16)] = jnp.zeros((16,), jnp.float32)
        return carry

    lax.fori_loop(0, C, zrow, 0)
    stripe = s * RPS
    for i in range(RPS // C):
        pltpu.sync_copy(ms0, acc.at[pl.ds(stripe + i * C, C)])
    plsc.subcore_barrier()

    ebase = s * EPS

    def gather_start(j, r, rbuf):
        pltpu.async_copy(featb2.at[rv.at[pl.ds(j * C, C)]], rbuf, semg.at[r])

    def gather_wait(j, r, rbuf):
        pltpu.make_async_copy(
            featb2.at[rv.at[pl.ds(j * C, C)]], rbuf, semg.at[r]
        ).wait()

    def colw_start(j, r, cbuf, wbuf):
        pltpu.async_copy(colp.at[pl.ds(ebase + j * C, C)], cbuf, semc.at[r])
        pltpu.async_copy(ewp.at[pl.ds(ebase + j * C, C)], wbuf, semw.at[r])

    def colw_wait(j, r, cbuf, wbuf):
        pltpu.make_async_copy(
            colp.at[pl.ds(ebase + j * C, C)], cbuf, semc.at[r]
        ).wait()
        pltpu.make_async_copy(
            ewp.at[pl.ds(ebase + j * C, C)], wbuf, semw.at[r]
        ).wait()

    def scatter_start(r, mbuf, cbuf):
        pltpu.async_copy(mbuf, acc.at[cbuf], sems.at[r], add=True)

    def scatter_wait(r, mbuf, cbuf):
        pltpu.make_async_copy(mbuf, acc.at[cbuf], sems.at[r]).wait()

    colw_start(0, 0, col0, wv0)
    gather_start(0, 0, rb0)

    def do_slot(j, b):
        rbuf, mbuf, cbuf, wbuf = rings[b]
        nb = (b + 1) % 3
        nrbuf, nmbuf, ncbuf, nwbuf = rings[nb]

        @pl.when(j >= 2)
        def _():
            scatter_wait(nb, nmbuf, ncbuf)

        @pl.when(j + 1 < NCHUNK)
        def _():
            colw_start(j + 1, nb, ncbuf, nwbuf)
            gather_start(j + 1, nb, nrbuf)

        gather_wait(j, b, rbuf)
        colw_wait(j, b, cbuf, wbuf)

        def emul(g, carry2):
            wvec = wbuf[pl.ds(g * 16, 16)]
            himask = jnp.full((16,), -65536, jnp.int32)
            for lane in range(16):
                bidx = jnp.full((16,), lane, jnp.int32)
                wb = wvec.at[bidx].get(mode="promise_in_bounds")
                e = g * 16 + lane
                for t in range(H // 32):
                    pi = rbuf[e, pl.ds(t * 16, 16)]
                    lo = lax.bitcast_convert_type(pi << 16, jnp.float32)
                    hi = lax.bitcast_convert_type(pi & himask, jnp.float32)
                    mbuf[e, pl.ds(t * 32, 16)] = lo * wb
                    mbuf[e, pl.ds(t * 32 + 16, 16)] = hi * wb
            return carry2

        lax.fori_loop(0, GROUPS, emul, 0)

        scatter_start(b, mbuf, cbuf)

    def triple(p, carry):
        j0 = p * 3
        do_slot(j0, 0)
        do_slot(j0 + 1, 1)
        do_slot(j0 + 2, 2)
        return carry

    lax.fori_loop(0, NCHUNK // 3, triple, 0)
    scatter_wait(1, ms1, col1)
    scatter_wait(2, ms2, col2)
    plsc.subcore_barrier()

    for i in range(RPS // C):
        r0 = stripe + i * C
        pltpu.sync_copy(
            acc.at[pl.ds(r0, C)], g_out.at[pl.ds(r0, C), pl.ds(c * H, H)]
        )


_sc_scatter = functools.partial(
    pl.kernel,
    mesh=plsc.VectorSubcoreMesh(core_axis_name="c", subcore_axis_name="s"),
    out_type=jax.ShapeDtypeStruct((N_PAD, F), jnp.float32),
    compiler_params=pltpu.CompilerParams(use_tc_tiling_on_sc=False),
    scratch_types=[
        pltpu.VMEM((EPS,), jnp.int32),
        pltpu.VMEM((C, H // 2), jnp.int32),
        pltpu.VMEM((C, H // 2), jnp.int32),
        pltpu.VMEM((C, H // 2), jnp.int32),
        pltpu.VMEM((C, H), jnp.float32),
        pltpu.VMEM((C, H), jnp.float32),
        pltpu.VMEM((C, H), jnp.float32),
        pltpu.VMEM((C,), jnp.int32),
        pltpu.VMEM((C,), jnp.int32),
        pltpu.VMEM((C,), jnp.int32),
        pltpu.VMEM((C,), jnp.float32),
        pltpu.VMEM((C,), jnp.float32),
        pltpu.VMEM((C,), jnp.float32),
        pltpu.VMEM_SHARED((N_PAD, H), jnp.float32),
        pltpu.SemaphoreType.DMA((3,)),
        pltpu.SemaphoreType.DMA((3,)),
        pltpu.SemaphoreType.DMA((3,)),
        pltpu.SemaphoreType.DMA((3,)),
    ],
)(_sc_body)


def _pack_body(f_ref, o_ref):
    x = f_ref[...]
    xi = lax.bitcast_convert_type(x, jnp.int32)
    bf = ((xi + 32767 + ((xi >> 16) & 1)) >> 16) & 65535
    for h in range(2):
        half = bf[:, h * H:(h + 1) * H]
        lo = half[:, : H // 2]
        hi = half[:, H // 2:]
        o_ref[h] = lo | (hi << 16)


def _pack_feat(feat):
    return pl.pallas_call(
        _pack_body,
        grid=(10,),
        in_specs=[pl.BlockSpec((N_NODES // 10, F), lambda i: (i, 0))],
        out_specs=pl.BlockSpec((2, N_NODES // 10, H // 2), lambda i: (0, i, 0)),
        out_shape=jax.ShapeDtypeStruct((2, N_NODES, H // 2), jnp.int32),
    )(feat)


def _mm_body(g_ref, wt_ref, a_ref, o_ref):
    x = jnp.dot(g_ref[...], wt_ref[...], preferred_element_type=jnp.float32)
    a = a_ref[0]
    o_ref[...] = jnp.where(x >= 0.0, x, a * x)


def _matmul_prelu(g, wt, a_arr):
    return pl.pallas_call(
        _mm_body,
        grid=(N_PAD // 1024,),
        in_specs=[
            pl.BlockSpec((1024, F), lambda i: (i, 0)),
            pl.BlockSpec((F, F), lambda i: (0, 0)),
            pl.BlockSpec(memory_space=pltpu.SMEM),
        ],
        out_specs=pl.BlockSpec((1024, F), lambda i: (i, 0)),
        out_shape=jax.ShapeDtypeStruct((N_PAD, F), jnp.float32),
    )(g, wt, a_arr)


@jax.jit
def kernel(feat, edge_index, edge_weight, W, prelu_a):
    row = edge_index[0].astype(jnp.int32)
    col = edge_index[1].astype(jnp.int32)
    zpad = jnp.zeros((E_PAD - E,), jnp.int32)
    rowp = jnp.concatenate([row, zpad])
    colp = jnp.concatenate([col, zpad])
    ewp = jnp.concatenate([edge_weight, jnp.zeros((E_PAD - E,), jnp.float32)])
    rows2 = jnp.stack([rowp, rowp + N_NODES])
    featb2 = _pack_feat(feat).reshape(2 * N_NODES, H // 2)

    g = _sc_scatter(featb2, rows2, colp, ewp)

    wt_perm = W.T[jnp.asarray(_PERM), :]
    out = _matmul_prelu(g, wt_perm, prelu_a.reshape(1))
    return out[:N_NODES]


# speedup vs baseline: 1.5561x; 1.1642x over previous
import functools

import jax
import jax.numpy as jnp
import numpy as np
from jax import lax
from jax.experimental import pallas as pl
from jax.experimental.pallas import tpu as pltpu
from jax.experimental.pallas import tpu_sc as plsc

N_NODES = 10000
E = 160000
F = 256
H = 128
NC = 2
NS = 16
C = 64
NCHUNK = 162
EPS = NCHUNK * C
E_PAD = NS * EPS
N_PAD = 10240
RPS = N_PAD // NS
GROUPS = C // 16

_PERM = np.empty((F,), np.int64)
for _h in range(2):
    for _t in range(4):
        for _i in range(16):
            _PERM[_h * 128 + _t * 32 + _i] = _h * 128 + _t * 16 + _i
            _PERM[_h * 128 + _t * 32 + 16 + _i] = _h * 128 + _t * 16 + _i + 64


def _sc_body(featb2, rows2, colp, ewp, g_out,
             rv, rb0, rb1, rb2, ms0, ms1, ms2, col0, col1, col2,
             wv0, wv1, wv2, acc, semg, sems, semc, semw):
    c = lax.axis_index("c")
    s = lax.axis_index("s")
    rings = (
        (rb0, ms0, col0, wv0),
        (rb1, ms1, col1, wv1),
        (rb2, ms2, col2, wv2),
    )

    pltpu.sync_copy(rows2.at[c, pl.ds(s * EPS, EPS)], rv)

    def zrow(i, carry):
        for t in range(H // 16):
            ms0[i, pl.ds(t * 16, 16)] = jnp.zeros((16,), jnp.float32)
        return carry

    lax.fori_loop(0, C, zrow, 0)
    stripe = s * RPS
    for i in range(RPS // C):
        pltpu.sync_copy(ms0, acc.at[pl.ds(stripe + i * C, C)])
    plsc.subcore_barrier()

    ebase = s * EPS

    def gather_start(j, r, rbuf):
        pltpu.async_copy(featb2.at[rv.at[pl.ds(j * C, C)]], rbuf, semg.at[r])

    def gather_wait(j, r, rbuf):
        pltpu.make_async_copy(
            featb2.at[rv.at[pl.ds(j * C, C)]], rbuf, semg.at[r]
        ).wait()

    def colw_start(j, r, cbuf, wbuf):
        pltpu.async_copy(colp.at[pl.ds(ebase + j * C, C)], cbuf, semc.at[r])
        pltpu.async_copy(ewp.at[pl.ds(ebase + j * C, C)], wbuf, semw.at[r])

    def colw_wait(j, r, cbuf, wbuf):
        pltpu.make_async_copy(
            colp.at[pl.ds(ebase + j * C, C)], cbuf, semc.at[r]
        ).wait()
        pltpu.make_async_copy(
            ewp.at[pl.ds(ebase + j * C, C)], wbuf, semw.at[r]
        ).wait()

    def scatter_start(r, mbuf, cbuf):
        pltpu.async_copy(mbuf, acc.at[cbuf], sems.at[r], add=True)

    def scatter_wait(r, mbuf, cbuf):
        pltpu.make_async_copy(mbuf, acc.at[cbuf], sems.at[r]).wait()

    colw_start(0, 0, col0, wv0)
    gather_start(0, 0, rb0)

    def do_slot(j, b):
        rbuf, mbuf, cbuf, wbuf = rings[b]
        nb = (b + 1) % 3
        nrbuf, nmbuf, ncbuf, nwbuf = rings[nb]

        @pl.when(j >= 2)
        def _():
            scatter_wait(nb, nmbuf, ncbuf)

        @pl.when(j + 1 < NCHUNK)
        def _():
            colw_start(j + 1, nb, ncbuf, nwbuf)
            gather_start(j + 1, nb, nrbuf)

        gather_wait(j, b, rbuf)
        colw_wait(j, b, cbuf, wbuf)

        def emul(g, carry2):
            wvec = wbuf[pl.ds(g * 16, 16)]
            himask = jnp.full((16,), -65536, jnp.int32)
            for lane in range(16):
                bidx = jnp.full((16,), lane, jnp.int32)
                wb = wvec.at[bidx].get(mode="promise_in_bounds")
                e = g * 16 + lane
                for t in range(H // 32):
                    pi = rbuf[e, pl.ds(t * 16, 16)]
                    lo = lax.bitcast_convert_type(pi << 16, jnp.float32)
                    hi = lax.bitcast_convert_type(pi & himask, jnp.float32)
                    mbuf[e, pl.ds(t * 32, 16)] = lo * wb
                    mbuf[e, pl.ds(t * 32 + 16, 16)] = hi * wb
            return carry2

        @plsc.parallel_loop(0, GROUPS, unroll=2)
        def _(g):
            emul(g, 0)

        scatter_start(b, mbuf, cbuf)

    def triple(p, carry):
        j0 = p * 3
        do_slot(j0, 0)
        do_slot(j0 + 1, 1)
        do_slot(j0 + 2, 2)
        return carry

    lax.fori_loop(0, NCHUNK // 3, triple, 0)
    scatter_wait(1, ms1, col1)
    scatter_wait(2, ms2, col2)
    plsc.subcore_barrier()

    for i in range(RPS // C):
        r0 = stripe + i * C
        pltpu.sync_copy(
            acc.at[pl.ds(r0, C)], g_out.at[pl.ds(r0, C), pl.ds(c * H, H)]
        )


_sc_scatter = functools.partial(
    pl.kernel,
    mesh=plsc.VectorSubcoreMesh(core_axis_name="c", subcore_axis_name="s"),
    out_type=jax.ShapeDtypeStruct((N_PAD, F), jnp.float32),
    compiler_params=pltpu.CompilerParams(use_tc_tiling_on_sc=False),
    scratch_types=[
        pltpu.VMEM((EPS,), jnp.int32),
        pltpu.VMEM((C, H // 2), jnp.int32),
        pltpu.VMEM((C, H // 2), jnp.int32),
        pltpu.VMEM((C, H // 2), jnp.int32),
        pltpu.VMEM((C, H), jnp.float32),
        pltpu.VMEM((C, H), jnp.float32),
        pltpu.VMEM((C, H), jnp.float32),
        pltpu.VMEM((C,), jnp.int32),
        pltpu.VMEM((C,), jnp.int32),
        pltpu.VMEM((C,), jnp.int32),
        pltpu.VMEM((C,), jnp.float32),
        pltpu.VMEM((C,), jnp.float32),
        pltpu.VMEM((C,), jnp.float32),
        pltpu.VMEM_SHARED((N_PAD, H), jnp.float32),
        pltpu.SemaphoreType.DMA((3,)),
        pltpu.SemaphoreType.DMA((3,)),
        pltpu.SemaphoreType.DMA((3,)),
        pltpu.SemaphoreType.DMA((3,)),
    ],
)(_sc_body)


def _pack_body(f_ref, o_ref):
    x = f_ref[...]
    xi = lax.bitcast_convert_type(x, jnp.int32)
    bf = ((xi + 32767 + ((xi >> 16) & 1)) >> 16) & 65535
    for h in range(2):
        half = bf[:, h * H:(h + 1) * H]
        lo = half[:, : H // 2]
        hi = half[:, H // 2:]
        o_ref[h] = lo | (hi << 16)


def _pack_feat(feat):
    return pl.pallas_call(
        _pack_body,
        grid=(10,),
        in_specs=[pl.BlockSpec((N_NODES // 10, F), lambda i: (i, 0))],
        out_specs=pl.BlockSpec((2, N_NODES // 10, H // 2), lambda i: (0, i, 0)),
        out_shape=jax.ShapeDtypeStruct((2, N_NODES, H // 2), jnp.int32),
    )(feat)


def _mm_body(g_ref, wt_ref, a_ref, o_ref):
    x = jnp.dot(g_ref[...], wt_ref[...], preferred_element_type=jnp.float32)
    a = a_ref[0]
    o_ref[...] = jnp.where(x >= 0.0, x, a * x)


def _matmul_prelu(g, wt, a_arr):
    return pl.pallas_call(
        _mm_body,
        grid=(N_PAD // 1024,),
        in_specs=[
            pl.BlockSpec((1024, F), lambda i: (i, 0)),
            pl.BlockSpec((F, F), lambda i: (0, 0)),
            pl.BlockSpec(memory_space=pltpu.SMEM),
        ],
        out_specs=pl.BlockSpec((1024, F), lambda i: (i, 0)),
        out_shape=jax.ShapeDtypeStruct((N_PAD, F), jnp.float32),
    )(g, wt, a_arr)


@jax.jit
def kernel(feat, edge_index, edge_weight, W, prelu_a):
    row = edge_index[0].astype(jnp.int32)
    col = edge_index[1].astype(jnp.int32)
    zpad = jnp.zeros((E_PAD - E,), jnp.int32)
    rowp = jnp.concatenate([row, zpad])
    colp = jnp.concatenate([col, zpad])
    ewp = jnp.concatenate([edge_weight, jnp.zeros((E_PAD - E,), jnp.float32)])
    rows2 = jnp.stack([rowp, rowp + N_NODES])
    featb2 = _pack_feat(feat).reshape(2 * N_NODES, H // 2)

    g = _sc_scatter(featb2, rows2, colp, ewp)

    wt_perm = W.T[jnp.asarray(_PERM), :]
    out = _matmul_prelu(g, wt_perm, prelu_a.reshape(1))
    return out[:N_NODES]


# speedup vs baseline: 1.6578x; 1.0653x over previous
import functools

import jax
import jax.numpy as jnp
import numpy as np
from jax import lax
from jax.experimental import pallas as pl
from jax.experimental.pallas import tpu as pltpu
from jax.experimental.pallas import tpu_sc as plsc

N_NODES = 10000
E = 160000
F = 256
H = 128
NC = 2
NS = 16
C = 64
NCHUNK = 162
EPS = NCHUNK * C
E_PAD = NS * EPS
N_PAD = 10240
RPS = N_PAD // NS
GROUPS = C // 16

_PERM = np.empty((F,), np.int64)
for _h in range(2):
    for _t in range(4):
        for _i in range(16):
            _PERM[_h * 128 + _t * 32 + _i] = _h * 128 + _t * 16 + _i
            _PERM[_h * 128 + _t * 32 + 16 + _i] = _h * 128 + _t * 16 + _i + 64


def _sc_body(featb2, rows2, colp, ewp, g_out,
             rv, rb0, rb1, rb2, ms0, ms1, ms2, col0, col1, col2,
             wv0, wv1, wv2, acc, semg, sems, semc, semw):
    c = lax.axis_index("c")
    s = lax.axis_index("s")
    rings = (
        (rb0, ms0, col0, wv0),
        (rb1, ms1, col1, wv1),
        (rb2, ms2, col2, wv2),
    )

    pltpu.sync_copy(rows2.at[c, pl.ds(s * EPS, EPS)], rv)

    def zrow(i, carry):
        for t in range(H // 16):
            ms0[i, pl.ds(t * 16, 16)] = jnp.zeros((16,), jnp.float32)
        return carry

    lax.fori_loop(0, C, zrow, 0)
    stripe = s * RPS
    for i in range(RPS // C):
        pltpu.sync_copy(ms0, acc.at[pl.ds(stripe + i * C, C)])
    plsc.subcore_barrier()

    ebase = s * EPS

    def gather_start(j, r, rbuf):
        pltpu.async_copy(featb2.at[rv.at[pl.ds(j * C, C)]], rbuf, semg.at[r])

    def gather_wait(j, r, rbuf):
        pltpu.make_async_copy(
            featb2.at[rv.at[pl.ds(j * C, C)]], rbuf, semg.at[r]
        ).wait()

    def colw_start(j, r, cbuf, wbuf):
        pltpu.async_copy(colp.at[pl.ds(ebase + j * C, C)], cbuf, semc.at[r])
        pltpu.async_copy(ewp.at[pl.ds(ebase + j * C, C)], wbuf, semw.at[r])

    def colw_wait(j, r, cbuf, wbuf):
        pltpu.make_async_copy(
            colp.at[pl.ds(ebase + j * C, C)], cbuf, semc.at[r]
        ).wait()
        pltpu.make_async_copy(
            ewp.at[pl.ds(ebase + j * C, C)], wbuf, semw.at[r]
        ).wait()

    def scatter_start(r, mbuf, cbuf):
        pltpu.async_copy(mbuf, acc.at[cbuf], sems.at[r], add=True)

    def scatter_wait(r, mbuf, cbuf):
        pltpu.make_async_copy(mbuf, acc.at[cbuf], sems.at[r]).wait()

    colw_start(0, 0, col0, wv0)
    gather_start(0, 0, rb0)

    def do_slot(j, b):
        rbuf, mbuf, cbuf, wbuf = rings[b]
        nb = (b + 1) % 3
        nrbuf, nmbuf, ncbuf, nwbuf = rings[nb]

        @pl.when(j >= 2)
        def _():
            scatter_wait(nb, nmbuf, ncbuf)

        @pl.when(j + 1 < NCHUNK)
        def _():
            colw_start(j + 1, nb, ncbuf, nwbuf)
            gather_start(j + 1, nb, nrbuf)

        gather_wait(j, b, rbuf)
        colw_wait(j, b, cbuf, wbuf)

        def emul(g, carry2):
            wvec = wbuf[pl.ds(g * 16, 16)]
            himask = jnp.full((16,), -65536, jnp.int32)
            for lane in range(16):
                bidx = jnp.full((16,), lane, jnp.int32)
                wb = wvec.at[bidx].get(mode="promise_in_bounds")
                e = g * 16 + lane
                for t in range(H // 32):
                    pi = rbuf[e, pl.ds(t * 16, 16)]
                    lo = lax.bitcast_convert_type(pi << 16, jnp.float32)
                    hi = lax.bitcast_convert_type(pi & himask, jnp.float32)
                    mbuf[e, pl.ds(t * 32, 16)] = lo * wb
                    mbuf[e, pl.ds(t * 32 + 16, 16)] = hi * wb
            return carry2

        @plsc.parallel_loop(0, GROUPS, unroll=4)
        def _(g):
            emul(g, 0)

        scatter_start(b, mbuf, cbuf)

    def triple(p, carry):
        j0 = p * 3
        do_slot(j0, 0)
        do_slot(j0 + 1, 1)
        do_slot(j0 + 2, 2)
        return carry

    lax.fori_loop(0, NCHUNK // 3, triple, 0)
    scatter_wait(1, ms1, col1)
    scatter_wait(2, ms2, col2)
    plsc.subcore_barrier()

    for i in range(RPS // C):
        r0 = stripe + i * C
        pltpu.sync_copy(
            acc.at[pl.ds(r0, C)], g_out.at[pl.ds(r0, C), pl.ds(c * H, H)]
        )


_sc_scatter = functools.partial(
    pl.kernel,
    mesh=plsc.VectorSubcoreMesh(core_axis_name="c", subcore_axis_name="s"),
    out_type=jax.ShapeDtypeStruct((N_PAD, F), jnp.float32),
    compiler_params=pltpu.CompilerParams(use_tc_tiling_on_sc=False),
    scratch_types=[
        pltpu.VMEM((EPS,), jnp.int32),
        pltpu.VMEM((C, H // 2), jnp.int32),
        pltpu.VMEM((C, H // 2), jnp.int32),
        pltpu.VMEM((C, H // 2), jnp.int32),
        pltpu.VMEM((C, H), jnp.float32),
        pltpu.VMEM((C, H), jnp.float32),
        pltpu.VMEM((C, H), jnp.float32),
        pltpu.VMEM((C,), jnp.int32),
        pltpu.VMEM((C,), jnp.int32),
        pltpu.VMEM((C,), jnp.int32),
        pltpu.VMEM((C,), jnp.float32),
        pltpu.VMEM((C,), jnp.float32),
        pltpu.VMEM((C,), jnp.float32),
        pltpu.VMEM_SHARED((N_PAD, H), jnp.float32),
        pltpu.SemaphoreType.DMA((3,)),
        pltpu.SemaphoreType.DMA((3,)),
        pltpu.SemaphoreType.DMA((3,)),
        pltpu.SemaphoreType.DMA((3,)),
    ],
)(_sc_body)


def _pack_body(f_ref, o_ref):
    x = f_ref[...]
    xi = lax.bitcast_convert_type(x, jnp.int32)
    bf = ((xi + 32767 + ((xi >> 16) & 1)) >> 16) & 65535
    for h in range(2):
        half = bf[:, h * H:(h + 1) * H]
        lo = half[:, : H // 2]
        hi = half[:, H // 2:]
        o_ref[h] = lo | (hi << 16)


def _pack_feat(feat):
    return pl.pallas_call(
        _pack_body,
        grid=(10,),
        in_specs=[pl.BlockSpec((N_NODES // 10, F), lambda i: (i, 0))],
        out_specs=pl.BlockSpec((2, N_NODES // 10, H // 2), lambda i: (0, i, 0)),
        out_shape=jax.ShapeDtypeStruct((2, N_NODES, H // 2), jnp.int32),
    )(feat)


def _mm_body(g_ref, wt_ref, a_ref, o_ref):
    x = jnp.dot(g_ref[...], wt_ref[...], preferred_element_type=jnp.float32)
    a = a_ref[0]
    o_ref[...] = jnp.where(x >= 0.0, x, a * x)


def _matmul_prelu(g, wt, a_arr):
    return pl.pallas_call(
        _mm_body,
        grid=(N_PAD // 1024,),
        in_specs=[
            pl.BlockSpec((1024, F), lambda i: (i, 0)),
            pl.BlockSpec((F, F), lambda i: (0, 0)),
            pl.BlockSpec(memory_space=pltpu.SMEM),
        ],
        out_specs=pl.BlockSpec((1024, F), lambda i: (i, 0)),
        out_shape=jax.ShapeDtypeStruct((N_PAD, F), jnp.float32),
    )(g, wt, a_arr)


@jax.jit
def kernel(feat, edge_index, edge_weight, W, prelu_a):
    row = edge_index[0].astype(jnp.int32)
    col = edge_index[1].astype(jnp.int32)
    zpad = jnp.zeros((E_PAD - E,), jnp.int32)
    rowp = jnp.concatenate([row, zpad])
    colp = jnp.concatenate([col, zpad])
    ewp = jnp.concatenate([edge_weight, jnp.zeros((E_PAD - E,), jnp.float32)])
    rows2 = jnp.stack([rowp, rowp + N_NODES])
    featb2 = _pack_feat(feat).reshape(2 * N_NODES, H // 2)

    g = _sc_scatter(featb2, rows2, colp, ewp)

    wt_perm = W.T[jnp.asarray(_PERM), :]
    out = _matmul_prelu(g, wt_perm, prelu_a.reshape(1))
    return out[:N_NODES]


# speedup vs baseline: 1.7149x; 1.0344x over previous
import functools

import jax
import jax.numpy as jnp
import numpy as np
from jax import lax
from jax.experimental import pallas as pl
from jax.experimental.pallas import tpu as pltpu
from jax.experimental.pallas import tpu_sc as plsc

N_NODES = 10000
E = 160000
F = 256
H = 128
NC = 2
NS = 16
C = 64
NCHUNK = 162
EPS = NCHUNK * C
E_PAD = NS * EPS
N_PAD = 10240
RPS = N_PAD // NS
GROUPS = C // 16

_PERM = np.empty((F,), np.int64)
for _h in range(2):
    for _t in range(4):
        for _i in range(16):
            _PERM[_h * 128 + _t * 32 + _i] = _h * 128 + _t * 16 + _i
            _PERM[_h * 128 + _t * 32 + 16 + _i] = _h * 128 + _t * 16 + _i + 64


def _sc_body(featb2, rows2, colp, ewp, g_out,
             rv, rb0, rb1, rb2, ms0, ms1, ms2, col0, col1, col2,
             wv0, wv1, wv2, acc, semg, sems, semc, semw, semz):
    c = lax.axis_index("c")
    s = lax.axis_index("s")
    rings = (
        (rb0, ms0, col0, wv0),
        (rb1, ms1, col1, wv1),
        (rb2, ms2, col2, wv2),
    )

    pltpu.sync_copy(rows2.at[c, pl.ds(s * EPS, EPS)], rv)

    def zrow(i, carry):
        for t in range(H // 16):
            ms0[i, pl.ds(t * 16, 16)] = jnp.zeros((16,), jnp.float32)
        return carry

    lax.fori_loop(0, C, zrow, 0)
    stripe = s * RPS
    for i in range(RPS // C):
        pltpu.async_copy(ms0, acc.at[pl.ds(stripe + i * C, C)], semz)
    for i in range(RPS // C):
        pltpu.make_async_copy(ms0, acc.at[pl.ds(stripe + i * C, C)], semz).wait()
    plsc.subcore_barrier()

    ebase = s * EPS

    def gather_start(j, r, rbuf):
        pltpu.async_copy(featb2.at[rv.at[pl.ds(j * C, C)]], rbuf, semg.at[r])

    def gather_wait(j, r, rbuf):
        pltpu.make_async_copy(
            featb2.at[rv.at[pl.ds(j * C, C)]], rbuf, semg.at[r]
        ).wait()

    def colw_start(j, r, cbuf, wbuf):
        pltpu.async_copy(colp.at[pl.ds(ebase + j * C, C)], cbuf, semc.at[r])
        pltpu.async_copy(ewp.at[pl.ds(ebase + j * C, C)], wbuf, semw.at[r])

    def colw_wait(j, r, cbuf, wbuf):
        pltpu.make_async_copy(
            colp.at[pl.ds(ebase + j * C, C)], cbuf, semc.at[r]
        ).wait()
        pltpu.make_async_copy(
            ewp.at[pl.ds(ebase + j * C, C)], wbuf, semw.at[r]
        ).wait()

    def scatter_start(r, mbuf, cbuf):
        pltpu.async_copy(mbuf, acc.at[cbuf], sems.at[r], add=True)

    def scatter_wait(r, mbuf, cbuf):
        pltpu.make_async_copy(mbuf, acc.at[cbuf], sems.at[r]).wait()

    colw_start(0, 0, col0, wv0)
    gather_start(0, 0, rb0)

    def do_slot(j, b):
        rbuf, mbuf, cbuf, wbuf = rings[b]
        nb = (b + 1) % 3
        nrbuf, nmbuf, ncbuf, nwbuf = rings[nb]

        @pl.when(j >= 2)
        def _():
            scatter_wait(nb, nmbuf, ncbuf)

        @pl.when(j + 1 < NCHUNK)
        def _():
            colw_start(j + 1, nb, ncbuf, nwbuf)
            gather_start(j + 1, nb, nrbuf)

        gather_wait(j, b, rbuf)
        colw_wait(j, b, cbuf, wbuf)

        def emul(g, carry2):
            wvec = wbuf[pl.ds(g * 16, 16)]
            himask = jnp.full((16,), -65536, jnp.int32)
            for lane in range(16):
                bidx = jnp.full((16,), lane, jnp.int32)
                wb = wvec.at[bidx].get(mode="promise_in_bounds")
                e = g * 16 + lane
                for t in range(H // 32):
                    pi = rbuf[e, pl.ds(t * 16, 16)]
                    lo = lax.bitcast_convert_type(pi << 16, jnp.float32)
                    hi = lax.bitcast_convert_type(pi & himask, jnp.float32)
                    mbuf[e, pl.ds(t * 32, 16)] = lo * wb
                    mbuf[e, pl.ds(t * 32 + 16, 16)] = hi * wb
            return carry2

        @plsc.parallel_loop(0, GROUPS, unroll=4)
        def _(g):
            emul(g, 0)

        scatter_start(b, mbuf, cbuf)

    def triple(p, carry):
        j0 = p * 3
        do_slot(j0, 0)
        do_slot(j0 + 1, 1)
        do_slot(j0 + 2, 2)
        return carry

    lax.fori_loop(0, NCHUNK // 3, triple, 0)
    scatter_wait(1, ms1, col1)
    scatter_wait(2, ms2, col2)
    plsc.subcore_barrier()

    for i in range(RPS // C):
        r0 = stripe + i * C
        pltpu.async_copy(
            acc.at[pl.ds(r0, C)], g_out.at[pl.ds(r0, C), pl.ds(c * H, H)], semz
        )
    for i in range(RPS // C):
        r0 = stripe + i * C
        pltpu.make_async_copy(
            acc.at[pl.ds(r0, C)], g_out.at[pl.ds(r0, C), pl.ds(c * H, H)], semz
        ).wait()


_sc_scatter = functools.partial(
    pl.kernel,
    mesh=plsc.VectorSubcoreMesh(core_axis_name="c", subcore_axis_name="s"),
    out_type=jax.ShapeDtypeStruct((N_PAD, F), jnp.float32),
    compiler_params=pltpu.CompilerParams(use_tc_tiling_on_sc=False),
    scratch_types=[
        pltpu.VMEM((EPS,), jnp.int32),
        pltpu.VMEM((C, H // 2), jnp.int32),
        pltpu.VMEM((C, H // 2), jnp.int32),
        pltpu.VMEM((C, H // 2), jnp.int32),
        pltpu.VMEM((C, H), jnp.float32),
        pltpu.VMEM((C, H), jnp.float32),
        pltpu.VMEM((C, H), jnp.float32),
        pltpu.VMEM((C,), jnp.int32),
        pltpu.VMEM((C,), jnp.int32),
        pltpu.VMEM((C,), jnp.int32),
        pltpu.VMEM((C,), jnp.float32),
        pltpu.VMEM((C,), jnp.float32),
        pltpu.VMEM((C,), jnp.float32),
        pltpu.VMEM_SHARED((N_PAD, H), jnp.float32),
        pltpu.SemaphoreType.DMA((3,)),
        pltpu.SemaphoreType.DMA((3,)),
        pltpu.SemaphoreType.DMA((3,)),
        pltpu.SemaphoreType.DMA((3,)),
        pltpu.SemaphoreType.DMA,
    ],
)(_sc_body)


def _pack_body(f_ref, o_ref):
    x = f_ref[...]
    xi = lax.bitcast_convert_type(x, jnp.int32)
    bf = ((xi + 32767 + ((xi >> 16) & 1)) >> 16) & 65535
    for h in range(2):
        half = bf[:, h * H:(h + 1) * H]
        lo = half[:, : H // 2]
        hi = half[:, H // 2:]
        o_ref[h] = lo | (hi << 16)


def _pack_feat(feat):
    return pl.pallas_call(
        _pack_body,
        grid=(10,),
        in_specs=[pl.BlockSpec((N_NODES // 10, F), lambda i: (i, 0))],
        out_specs=pl.BlockSpec((2, N_NODES // 10, H // 2), lambda i: (0, i, 0)),
        out_shape=jax.ShapeDtypeStruct((2, N_NODES, H // 2), jnp.int32),
    )(feat)


def _mm_body(g_ref, wt_ref, a_ref, o_ref):
    x = jnp.dot(g_ref[...], wt_ref[...], preferred_element_type=jnp.float32)
    a = a_ref[0]
    o_ref[...] = jnp.where(x >= 0.0, x, a * x)


def _matmul_prelu(g, wt, a_arr):
    return pl.pallas_call(
        _mm_body,
        grid=(10,),
        in_specs=[
            pl.BlockSpec((N_NODES // 10, F), lambda i: (i, 0)),
            pl.BlockSpec((F, F), lambda i: (0, 0)),
            pl.BlockSpec(memory_space=pltpu.SMEM),
        ],
        out_specs=pl.BlockSpec((N_NODES // 10, F), lambda i: (i, 0)),
        out_shape=jax.ShapeDtypeStruct((N_NODES, F), jnp.float32),
    )(g, wt, a_arr)


@jax.jit
def kernel(feat, edge_index, edge_weight, W, prelu_a):
    row = edge_index[0].astype(jnp.int32)
    col = edge_index[1].astype(jnp.int32)
    zpad = jnp.zeros((E_PAD - E,), jnp.int32)
    rowp = jnp.concatenate([row, zpad])
    colp = jnp.concatenate([col, zpad])
    ewp = jnp.concatenate([edge_weight, jnp.zeros((E_PAD - E,), jnp.float32)])
    rows2 = jnp.stack([rowp, rowp + N_NODES])
    featb2 = _pack_feat(feat).reshape(2 * N_NODES, H // 2)

    g = _sc_scatter(featb2, rows2, colp, ewp)

    wt_perm = W.T[jnp.asarray(_PERM), :]
    return _matmul_prelu(g, wt_perm, prelu_a.reshape(1))
